# Initial kernel scaffold; baseline (speedup 1.0000x reference)
import jax
import jax.numpy as jnp
from jax.experimental import pallas as pl


def kernel(atom_number, distance, atom_membership, distance_membership_i, distance_membership_j, embedding, W_cf, W_df, W_fc, b_cf, b_df, Wg0, bg0, Wg1, bg1, W_lin, b_lin):
    raise NotImplementedError("write your pallas kernel here")


# speedup vs baseline: 1.9756x; 1.9756x over previous
import functools

import jax
import jax.numpy as jnp
from jax import lax
from jax.experimental import pallas as pl
from jax.experimental.pallas import tpu as pltpu
from jax.experimental.pallas import tpu_sc as plsc

N_ATOMS = 50000
N_PAIRS = 800000
N_MOLS = 2500
N_DIST = 100

F32 = jnp.float32

NC = 2
NS = 16
NW = NC * NS

CH = 128
NCHUNK = N_PAIRS // CH
IDXPAD = 6256
KMAX_G = (NCHUNK + NW - 1) // NW
KMAX_S = (NCHUNK + NS - 1) // NS
HALF_A = N_ATOMS // NC
ACHUNK = 391
SENT = 0x7FFFFFFF

PBLK = 2000
ABLK = 2000

_CP_SC = pltpu.CompilerParams(use_tc_tiling_on_sc=False)


def _mesh():
    return plsc.VectorSubcoreMesh(core_axis_name="c", subcore_axis_name="s")


@functools.partial(
    pl.kernel,
    out_type=jax.ShapeDtypeStruct((N_PAIRS, 32), F32),
    mesh=_mesh(),
    compiler_params=_CP_SC,
    scratch_types=[
        pltpu.VMEM((1, CH), jnp.int32),
        pltpu.VMEM((CH, 32), F32),
        pltpu.SemaphoreType.DMA,
    ],
)
def _sc_gather32(table_hbm, idx_hbm, out_hbm, idxbuf, rowbuf, sem):
    wid = lax.axis_index("c") * NS + lax.axis_index("s")

    @pl.loop(0, KMAX_G)
    def _(kk):
        ch = wid + kk * NW

        @pl.when(ch < NCHUNK)
        def _():
            pltpu.sync_copy(idx_hbm.at[ch], idxbuf)
            pltpu.async_copy(table_hbm.at[idxbuf.at[0]], rowbuf, sem).wait()
            pltpu.sync_copy(rowbuf, out_hbm.at[pl.ds(ch * CH, CH)])


@functools.partial(
    pl.kernel,
    out_type=jax.ShapeDtypeStruct((N_ATOMS, 32), F32),
    mesh=_mesh(),
    compiler_params=_CP_SC,
    scratch_types=[
        pltpu.VMEM((1, CH), jnp.int32),
        pltpu.VMEM((1, CH), jnp.int32),
        pltpu.VMEM((CH, 32), F32),
        pltpu.VMEM((1000, 32), F32),
        pltpu.VMEM_SHARED((HALF_A, 32), F32),
    ],
)
def _sc_segsum32(u_hbm, idx_hbm, out_hbm, idxbuf, idxbuf2, ubuf, bigbuf, table):
    c_ax = lax.axis_index("c")
    sid = lax.axis_index("s")
    base = c_ax * HALF_A

    @pl.loop(0, 1000)
    def _(r):
        bigbuf[r, pl.ds(0, 16)] = jnp.zeros((16,), F32)
        bigbuf[r, pl.ds(16, 16)] = jnp.zeros((16,), F32)

    @pl.loop(0, 2)
    def _(kk):
        ch = sid + kk * NS

        @pl.when(ch < HALF_A // 1000)
        def _():
            pltpu.sync_copy(bigbuf, table.at[pl.ds(ch * 1000, 1000)])

    plsc.subcore_barrier()

    @pl.loop(0, KMAX_S)
    def _(kk):
        ch = sid + kk * NS

        @pl.when(ch < NCHUNK)
        def _():
            pltpu.sync_copy(idx_hbm.at[ch], idxbuf)
            for j in range(CH // 16):
                v = idxbuf[0, pl.ds(j * 16, 16)] - base
                ok = (v >= 0) & (v < HALF_A)
                idxbuf2[0, pl.ds(j * 16, 16)] = jnp.where(ok, v, SENT)
            pltpu.sync_copy(u_hbm.at[pl.ds(ch * CH, CH)], ubuf)
            pltpu.sync_copy(
                ubuf,
                table.at[plsc.Indices(idxbuf2.at[0], ignored_value=SENT)],
                add=True,
            )

    plsc.subcore_barrier()

    @pl.loop(0, 2)
    def _(kk):
        ch = sid + kk * NS

        @pl.when(ch < HALF_A // 1000)
        def _():
            pltpu.sync_copy(table.at[pl.ds(ch * 1000, 1000)], bigbuf)
            pltpu.sync_copy(bigbuf, out_hbm.at[pl.ds(base + ch * 1000, 1000)])


@functools.partial(
    pl.kernel,
    out_type=jax.ShapeDtypeStruct((NC, N_MOLS, 16), F32),
    mesh=_mesh(),
    compiler_params=_CP_SC,
    scratch_types=[
        pltpu.VMEM((1, CH), jnp.int32),
        pltpu.VMEM((CH, 16), F32),
        pltpu.VMEM((1000, 16), F32),
        pltpu.VMEM_SHARED((N_MOLS, 16), F32),
    ],
)
def _sc_molsum16(h_hbm, idx_hbm, out_hbm, idxbuf, ubuf, bigbuf, table):
    c_ax = lax.axis_index("c")
    sid = lax.axis_index("s")

    @pl.loop(0, 1000)
    def _(r):
        bigbuf[r, pl.ds(0, 16)] = jnp.zeros((16,), F32)

    @pl.loop(0, 3)
    def _(kk):
        ch = sid + kk * NS

        @pl.when(ch < 2)
        def _():
            pltpu.sync_copy(bigbuf, table.at[pl.ds(ch * 1000, 1000)])

        @pl.when(ch == 2)
        def _():
            pltpu.sync_copy(
                bigbuf.at[pl.ds(0, 500)], table.at[pl.ds(2000, 500)]
            )

    plsc.subcore_barrier()

    @pl.loop(0, (ACHUNK + NW - 1) // NW + 1)
    def _(kk):
        ch = (sid + kk * NS) * NC + c_ax

        @pl.when(ch < ACHUNK - 1)
        def _():
            pltpu.sync_copy(idx_hbm.at[ch], idxbuf)
            pltpu.sync_copy(h_hbm.at[pl.ds(ch * CH, CH)], ubuf)
            pltpu.sync_copy(
                ubuf,
                table.at[plsc.Indices(idxbuf.at[0], ignored_value=SENT)],
                add=True,
            )

        @pl.when(ch == ACHUNK - 1)
        def _():
            pltpu.sync_copy(idx_hbm.at[ch], idxbuf)
            pltpu.sync_copy(
                h_hbm.at[pl.ds(ch * CH, 80)], ubuf.at[pl.ds(0, 80)]
            )
            pltpu.sync_copy(
                ubuf,
                table.at[plsc.Indices(idxbuf.at[0], ignored_value=SENT)],
                add=True,
            )

    plsc.subcore_barrier()

    @pl.loop(0, 3)
    def _(kk):
        ch = sid + kk * NS

        @pl.when(ch < 2)
        def _():
            pltpu.sync_copy(table.at[pl.ds(ch * 1000, 1000)], bigbuf)
            pltpu.sync_copy(bigbuf, out_hbm.at[c_ax, pl.ds(ch * 1000, 1000)])

        @pl.when(ch == 2)
        def _():
            pltpu.sync_copy(
                table.at[pl.ds(2000, 500)], bigbuf.at[pl.ds(0, 500)]
            )
            pltpu.sync_copy(
                bigbuf.at[pl.ds(0, 500)], out_hbm.at[c_ax, pl.ds(2000, 500)]
            )


def _tc_x1_body(an_ref, emb_ref, out_ref):
    an = an_ref[0, 0, :]
    oh = (
        lax.broadcasted_iota(jnp.int32, (ABLK, 32), 1) == an[:, None]
    ).astype(F32)
    out_ref[...] = jnp.dot(oh, emb_ref[...], preferred_element_type=F32)


def _tc_pairs_body(d_ref, g_ref, wdf_ref, bdf_ref, wcf_ref, bcf_ref, wfc_ref, out_ref):
    dh = jnp.dot(d_ref[...], wdf_ref[...], preferred_element_type=F32) + bdf_ref[...]
    afh = jnp.dot(g_ref[...], wcf_ref[...], preferred_element_type=F32) + bcf_ref[...]
    out_ref[...] = jnp.tanh(
        jnp.dot(dh * afh, wfc_ref[...], preferred_element_type=F32)
    )


def _tc_update_body(p_ref, x_ref, wcf_ref, bcf_ref, bdf_ref, wfc_ref, out_ref):
    x = x_ref[...]
    afh = jnp.dot(x, wcf_ref[...], preferred_element_type=F32) + bcf_ref[...]
    oii = jnp.tanh(
        jnp.dot(bdf_ref[...] * afh, wfc_ref[...], preferred_element_type=F32)
    )
    out_ref[...] = p_ref[...] - oii + x


def _tc_head_body(
    p_ref, x_ref, wcf_ref, bcf_ref, bdf_ref, wfc_ref,
    wg0_ref, bg0_ref, wg1_ref, bg1_ref, out_ref,
):
    x = x_ref[...]
    afh = jnp.dot(x, wcf_ref[...], preferred_element_type=F32) + bcf_ref[...]
    oii = jnp.tanh(
        jnp.dot(bdf_ref[...] * afh, wfc_ref[...], preferred_element_type=F32)
    )
    x3 = p_ref[...] - oii + x
    h = jnp.tanh(jnp.dot(x3, wg0_ref[...], preferred_element_type=F32) + bg0_ref[...])
    out_ref[...] = jnp.tanh(
        jnp.dot(h, wg1_ref[...], preferred_element_type=F32) + bg1_ref[...]
    )


def _tc_final_body(pm_ref, sel_ref, b_ref, out_ref):
    s = pm_ref[0] + pm_ref[1]
    out_ref[...] = jnp.dot(s, sel_ref[...], preferred_element_type=F32) + b_ref[...]


def _full(shape):
    return pl.BlockSpec(shape, lambda *_: tuple(0 for _ in shape))


def _tc_x1(an3, emb_p):
    return pl.pallas_call(
        _tc_x1_body,
        grid=(N_ATOMS // ABLK,),
        in_specs=[
            pl.BlockSpec((1, 1, ABLK), lambda i: (i, 0, 0)),
            _full((32, 32)),
        ],
        out_specs=pl.BlockSpec((ABLK, 32), lambda i: (i, 0)),
        out_shape=jax.ShapeDtypeStruct((N_ATOMS, 32), F32),
    )(an3, emb_p)


def _tc_pairs(distance, g, wdf, bdf, wcf, bcf, wfc):
    return pl.pallas_call(
        _tc_pairs_body,
        grid=(N_PAIRS // PBLK,),
        in_specs=[
            pl.BlockSpec((PBLK, N_DIST), lambda i: (i, 0)),
            pl.BlockSpec((PBLK, 32), lambda i: (i, 0)),
            _full((N_DIST, 64)),
            _full((1, 64)),
            _full((32, 64)),
            _full((1, 64)),
            _full((64, 32)),
        ],
        out_specs=pl.BlockSpec((PBLK, 32), lambda i: (i, 0)),
        out_shape=jax.ShapeDtypeStruct((N_PAIRS, 32), F32),
    )(distance, g, wdf, bdf, wcf, bcf, wfc)


def _tc_update(p, x, wcf, bcf, bdf, wfc):
    return pl.pallas_call(
        _tc_update_body,
        grid=(N_ATOMS // ABLK,),
        in_specs=[
            pl.BlockSpec((ABLK, 32), lambda i: (i, 0)),
            pl.BlockSpec((ABLK, 32), lambda i: (i, 0)),
            _full((32, 64)),
            _full((1, 64)),
            _full((1, 64)),
            _full((64, 32)),
        ],
        out_specs=pl.BlockSpec((ABLK, 32), lambda i: (i, 0)),
        out_shape=jax.ShapeDtypeStruct((N_ATOMS, 32), F32),
    )(p, x, wcf, bcf, bdf, wfc)


def _tc_head(p, x, wcf, bcf, bdf, wfc, wg0, bg0, wg1, bg1):
    return pl.pallas_call(
        _tc_head_body,
        grid=(N_ATOMS // ABLK,),
        in_specs=[
            pl.BlockSpec((ABLK, 32), lambda i: (i, 0)),
            pl.BlockSpec((ABLK, 32), lambda i: (i, 0)),
            _full((32, 64)),
            _full((1, 64)),
            _full((1, 64)),
            _full((64, 32)),
            _full((32, 128)),
            _full((1, 128)),
            _full((128, 16)),
            _full((1, 16)),
        ],
        out_specs=pl.BlockSpec((ABLK, 16), lambda i: (i, 0)),
        out_shape=jax.ShapeDtypeStruct((N_ATOMS, 16), F32),
    )(p, x, wcf, bcf, bdf, wfc, wg0, bg0, wg1, bg1)


def _tc_final(pm, sel, b):
    return pl.pallas_call(
        _tc_final_body,
        in_specs=[_full((NC, N_MOLS, 16)), _full((16, 1)), _full((1, 1))],
        out_specs=_full((N_MOLS, 1)),
        out_shape=jax.ShapeDtypeStruct((N_MOLS, 1), F32),
    )(pm, sel, b)


def kernel(atom_number, distance, atom_membership, distance_membership_i,
           distance_membership_j, embedding, W_cf, W_df, W_fc, b_cf, b_df,
           Wg0, bg0, Wg1, bg1, W_lin, b_lin):
    emb_p = jnp.zeros((32, 32), F32).at[:30, :30].set(embedding)
    wcf_p = jnp.zeros((32, 64), F32).at[:30, :60].set(W_cf)
    wdf_p = jnp.zeros((N_DIST, 64), F32).at[:, :60].set(W_df)
    wfc_p = jnp.zeros((64, 32), F32).at[:60, :30].set(W_fc)
    bcf_p = jnp.zeros((1, 64), F32).at[0, :60].set(b_cf)
    bdf_p = jnp.zeros((1, 64), F32).at[0, :60].set(b_df)
    wg0_p = jnp.zeros((32, 128), F32).at[:30, :100].set(Wg0)
    bg0_p = jnp.zeros((1, 128), F32).at[0, :100].set(bg0)
    wg1_p = jnp.zeros((128, 16), F32).at[:100, :1].set(Wg1)
    bg1_p = jnp.zeros((1, 16), F32).at[0, :1].set(bg1)
    sel = jnp.zeros((16, 1), F32).at[0, 0].set(W_lin[0, 0])
    b_fin = b_lin.reshape(1, 1).astype(F32)

    an3 = atom_number.astype(jnp.int32).reshape(N_ATOMS // ABLK, 1, ABLK)
    dmj_r = jnp.pad(
        distance_membership_j.astype(jnp.int32).reshape(NCHUNK, 1, CH),
        ((0, IDXPAD - NCHUNK), (0, 0), (0, 0)),
    )
    dmi_r = jnp.pad(
        distance_membership_i.astype(jnp.int32).reshape(NCHUNK, 1, CH),
        ((0, IDXPAD - NCHUNK), (0, 0), (0, 0)),
    )
    am_pad = jnp.pad(
        atom_membership.astype(jnp.int32), (0, ACHUNK * CH - N_ATOMS),
        constant_values=SENT,
    ).reshape(ACHUNK, 1, CH)
    am_r = jnp.pad(am_pad, ((0, 400 - ACHUNK), (0, 0), (0, 0)))

    x1 = _tc_x1(an3, emb_p)

    g1 = _sc_gather32(x1, dmj_r)
    u1 = _tc_pairs(distance, g1, wdf_p, bdf_p, wcf_p, bcf_p, wfc_p)
    s1 = _sc_segsum32(u1, dmi_r)
    x2 = _tc_update(s1, x1, wcf_p, bcf_p, bdf_p, wfc_p)

    g2 = _sc_gather32(x2, dmj_r)
    u2 = _tc_pairs(distance, g2, wdf_p, bdf_p, wcf_p, bcf_p, wfc_p)
    s2 = _sc_segsum32(u2, dmi_r)
    h2 = _tc_head(s2, x2, wcf_p, bcf_p, bdf_p, wfc_p,
                  wg0_p, bg0_p, wg1_p, bg1_p)

    pm = _sc_molsum16(h2, am_r)
    return _tc_final(pm, sel, b_fin)


# speedup vs baseline: 2.5251x; 1.2781x over previous
import functools

import jax
import jax.numpy as jnp
from jax import lax
from jax.experimental import pallas as pl
from jax.experimental.pallas import tpu as pltpu
from jax.experimental.pallas import tpu_sc as plsc

N_ATOMS = 50000
N_PAIRS = 800000
N_MOLS = 2500
N_DIST = 100

F32 = jnp.float32

NC = 2
NS = 16
NW = NC * NS

CH = 128
NCHUNK = N_PAIRS // CH
IDXPAD = 6256
KMAX_G = (NCHUNK + NW - 1) // NW
KMAX_S = (NCHUNK + NS - 1) // NS
HALF_A = N_ATOMS // NC
ACHUNK = 391
SENT = 0x7FFFFFFF

PBLK = 2000
ABLK = 2000

_CP_SC = pltpu.CompilerParams(use_tc_tiling_on_sc=False)


def _mesh():
    return plsc.VectorSubcoreMesh(core_axis_name="c", subcore_axis_name="s")


NBUF = 4
GMAX_G = (NCHUNK + NW * NBUF - 1) // (NW * NBUF)


@functools.partial(
    pl.kernel,
    out_type=jax.ShapeDtypeStruct((N_PAIRS, 32), F32),
    mesh=_mesh(),
    compiler_params=_CP_SC,
    scratch_types=[
        [pltpu.VMEM((1, CH), jnp.int32)] * NBUF,
        pltpu.VMEM((NBUF * CH, 32), F32),
        pltpu.SemaphoreType.DMA,
        pltpu.SemaphoreType.DMA,
        pltpu.SemaphoreType.DMA,
    ],
)
def _sc_gather32(table_hbm, idx_hbm, out_hbm, idxbuf, rowbuf, si, sg, so):
    wid = lax.axis_index("c") * NS + lax.axis_index("s")

    @pl.loop(0, GMAX_G)
    def _(g):
        chs = [wid + (g * NBUF + b) * NW for b in range(NBUF)]

        def _idx_copy(b):
            return pltpu.make_async_copy(idx_hbm.at[chs[b]], idxbuf[b], si)

        def _gat_copy(b):
            return pltpu.make_async_copy(
                table_hbm.at[idxbuf[b].at[0]],
                rowbuf.at[pl.ds(b * CH, CH)],
                sg,
            )

        def _out_copy(b):
            return pltpu.make_async_copy(
                rowbuf.at[pl.ds(b * CH, CH)],
                out_hbm.at[pl.ds(chs[b] * CH, CH)],
                so,
            )

        def _staged(mk):
            for b in range(NBUF):
                @pl.when(chs[b] < NCHUNK)
                def _(b=b):
                    mk(b).start()
            for b in range(NBUF):
                @pl.when(chs[b] < NCHUNK)
                def _(b=b):
                    mk(b).wait()

        _staged(_idx_copy)
        _staged(_gat_copy)
        _staged(_out_copy)


GMAX_S = (NCHUNK + NS * NBUF - 1) // (NS * NBUF)


@functools.partial(
    pl.kernel,
    out_type=jax.ShapeDtypeStruct((N_ATOMS, 32), F32),
    mesh=_mesh(),
    compiler_params=_CP_SC,
    scratch_types=[
        [pltpu.VMEM((1, CH), jnp.int32)] * NBUF,
        [pltpu.VMEM((1, CH), jnp.int32)] * NBUF,
        pltpu.VMEM((NBUF * CH, 32), F32),
        pltpu.VMEM((1000, 32), F32),
        pltpu.VMEM_SHARED((HALF_A, 32), F32),
        pltpu.SemaphoreType.DMA,
        pltpu.SemaphoreType.DMA,
        pltpu.SemaphoreType.DMA,
    ],
)
def _sc_segsum32(u_hbm, idx_hbm, out_hbm, idxbuf, idxbuf2, ubuf, bigbuf,
                 table, si, su, sa):
    c_ax = lax.axis_index("c")
    sid = lax.axis_index("s")
    base = c_ax * HALF_A

    @pl.loop(0, 1000)
    def _(r):
        bigbuf[r, pl.ds(0, 16)] = jnp.zeros((16,), F32)
        bigbuf[r, pl.ds(16, 16)] = jnp.zeros((16,), F32)

    @pl.loop(0, 2)
    def _(kk):
        ch = sid + kk * NS

        @pl.when(ch < HALF_A // 1000)
        def _():
            pltpu.sync_copy(bigbuf, table.at[pl.ds(ch * 1000, 1000)])

    plsc.subcore_barrier()

    @pl.loop(0, GMAX_S)
    def _(g):
        chs = [sid + (g * NBUF + b) * NS for b in range(NBUF)]

        def _idx_copy(b):
            return pltpu.make_async_copy(idx_hbm.at[chs[b]], idxbuf[b], si)

        def _u_copy(b):
            return pltpu.make_async_copy(
                u_hbm.at[pl.ds(chs[b] * CH, CH)],
                ubuf.at[pl.ds(b * CH, CH)],
                su,
            )

        scat_descs = [None] * NBUF

        def _scat_start(b):
            scat_descs[b] = pltpu.async_copy(
                ubuf.at[pl.ds(b * CH, CH)],
                table.at[plsc.Indices(idxbuf2[b].at[0], ignored_value=SENT)],
                sa,
                add=True,
            )

        for b in range(NBUF):
            @pl.when(chs[b] < NCHUNK)
            def _(b=b):
                _idx_copy(b).start()
                _u_copy(b).start()
        for b in range(NBUF):
            @pl.when(chs[b] < NCHUNK)
            def _(b=b):
                _idx_copy(b).wait()
        for b in range(NBUF):
            @pl.when(chs[b] < NCHUNK)
            def _(b=b):
                for j in range(CH // 16):
                    v = idxbuf[b][0, pl.ds(j * 16, 16)] - base
                    ok = (v >= 0) & (v < HALF_A)
                    idxbuf2[b][0, pl.ds(j * 16, 16)] = jnp.where(ok, v, SENT)
        for b in range(NBUF):
            @pl.when(chs[b] < NCHUNK)
            def _(b=b):
                _u_copy(b).wait()
        for b in range(NBUF):
            @pl.when(chs[b] < NCHUNK)
            def _(b=b):
                _scat_start(b)
        for b in range(NBUF):
            @pl.when(chs[b] < NCHUNK)
            def _(b=b):
                scat_descs[b].wait()

    plsc.subcore_barrier()

    @pl.loop(0, 2)
    def _(kk):
        ch = sid + kk * NS

        @pl.when(ch < HALF_A // 1000)
        def _():
            pltpu.sync_copy(table.at[pl.ds(ch * 1000, 1000)], bigbuf)
            pltpu.sync_copy(bigbuf, out_hbm.at[pl.ds(base + ch * 1000, 1000)])


@functools.partial(
    pl.kernel,
    out_type=jax.ShapeDtypeStruct((NC, N_MOLS, 16), F32),
    mesh=_mesh(),
    compiler_params=_CP_SC,
    scratch_types=[
        pltpu.VMEM((1, CH), jnp.int32),
        pltpu.VMEM((CH, 16), F32),
        pltpu.VMEM((1000, 16), F32),
        pltpu.VMEM_SHARED((N_MOLS, 16), F32),
    ],
)
def _sc_molsum16(h_hbm, idx_hbm, out_hbm, idxbuf, ubuf, bigbuf, table):
    c_ax = lax.axis_index("c")
    sid = lax.axis_index("s")

    @pl.loop(0, 1000)
    def _(r):
        bigbuf[r, pl.ds(0, 16)] = jnp.zeros((16,), F32)

    @pl.loop(0, 3)
    def _(kk):
        ch = sid + kk * NS

        @pl.when(ch < 2)
        def _():
            pltpu.sync_copy(bigbuf, table.at[pl.ds(ch * 1000, 1000)])

        @pl.when(ch == 2)
        def _():
            pltpu.sync_copy(
                bigbuf.at[pl.ds(0, 500)], table.at[pl.ds(2000, 500)]
            )

    plsc.subcore_barrier()

    @pl.loop(0, (ACHUNK + NW - 1) // NW + 1)
    def _(kk):
        ch = (sid + kk * NS) * NC + c_ax

        @pl.when(ch < ACHUNK - 1)
        def _():
            pltpu.sync_copy(idx_hbm.at[ch], idxbuf)
            pltpu.sync_copy(h_hbm.at[pl.ds(ch * CH, CH)], ubuf)
            pltpu.sync_copy(
                ubuf,
                table.at[plsc.Indices(idxbuf.at[0], ignored_value=SENT)],
                add=True,
            )

        @pl.when(ch == ACHUNK - 1)
        def _():
            pltpu.sync_copy(idx_hbm.at[ch], idxbuf)
            pltpu.sync_copy(
                h_hbm.at[pl.ds(ch * CH, 80)], ubuf.at[pl.ds(0, 80)]
            )
            pltpu.sync_copy(
                ubuf,
                table.at[plsc.Indices(idxbuf.at[0], ignored_value=SENT)],
                add=True,
            )

    plsc.subcore_barrier()

    @pl.loop(0, 3)
    def _(kk):
        ch = sid + kk * NS

        @pl.when(ch < 2)
        def _():
            pltpu.sync_copy(table.at[pl.ds(ch * 1000, 1000)], bigbuf)
            pltpu.sync_copy(bigbuf, out_hbm.at[c_ax, pl.ds(ch * 1000, 1000)])

        @pl.when(ch == 2)
        def _():
            pltpu.sync_copy(
                table.at[pl.ds(2000, 500)], bigbuf.at[pl.ds(0, 500)]
            )
            pltpu.sync_copy(
                bigbuf.at[pl.ds(0, 500)], out_hbm.at[c_ax, pl.ds(2000, 500)]
            )


def _tc_x1_body(an_ref, emb_ref, out_ref):
    an = an_ref[0, 0, :]
    oh = (
        lax.broadcasted_iota(jnp.int32, (ABLK, 32), 1) == an[:, None]
    ).astype(F32)
    out_ref[...] = jnp.dot(oh, emb_ref[...], preferred_element_type=F32)


def _tc_pairs_body(d_ref, g_ref, wdf_ref, bdf_ref, wcf_ref, bcf_ref, wfc_ref, out_ref):
    dh = jnp.dot(d_ref[...], wdf_ref[...], preferred_element_type=F32) + bdf_ref[...]
    afh = jnp.dot(g_ref[...], wcf_ref[...], preferred_element_type=F32) + bcf_ref[...]
    out_ref[...] = jnp.tanh(
        jnp.dot(dh * afh, wfc_ref[...], preferred_element_type=F32)
    )


def _tc_update_body(p_ref, x_ref, wcf_ref, bcf_ref, bdf_ref, wfc_ref, out_ref):
    x = x_ref[...]
    afh = jnp.dot(x, wcf_ref[...], preferred_element_type=F32) + bcf_ref[...]
    oii = jnp.tanh(
        jnp.dot(bdf_ref[...] * afh, wfc_ref[...], preferred_element_type=F32)
    )
    out_ref[...] = p_ref[...] - oii + x


def _tc_head_body(
    p_ref, x_ref, wcf_ref, bcf_ref, bdf_ref, wfc_ref,
    wg0_ref, bg0_ref, wg1_ref, bg1_ref, out_ref,
):
    x = x_ref[...]
    afh = jnp.dot(x, wcf_ref[...], preferred_element_type=F32) + bcf_ref[...]
    oii = jnp.tanh(
        jnp.dot(bdf_ref[...] * afh, wfc_ref[...], preferred_element_type=F32)
    )
    x3 = p_ref[...] - oii + x
    h = jnp.tanh(jnp.dot(x3, wg0_ref[...], preferred_element_type=F32) + bg0_ref[...])
    out_ref[...] = jnp.tanh(
        jnp.dot(h, wg1_ref[...], preferred_element_type=F32) + bg1_ref[...]
    )


def _tc_final_body(pm_ref, sel_ref, b_ref, out_ref):
    s = pm_ref[0] + pm_ref[1]
    out_ref[...] = jnp.dot(s, sel_ref[...], preferred_element_type=F32) + b_ref[...]


def _full(shape):
    return pl.BlockSpec(shape, lambda *_: tuple(0 for _ in shape))


def _tc_x1(an3, emb_p):
    return pl.pallas_call(
        _tc_x1_body,
        grid=(N_ATOMS // ABLK,),
        in_specs=[
            pl.BlockSpec((1, 1, ABLK), lambda i: (i, 0, 0)),
            _full((32, 32)),
        ],
        out_specs=pl.BlockSpec((ABLK, 32), lambda i: (i, 0)),
        out_shape=jax.ShapeDtypeStruct((N_ATOMS, 32), F32),
    )(an3, emb_p)


def _tc_pairs(distance, g, wdf, bdf, wcf, bcf, wfc):
    return pl.pallas_call(
        _tc_pairs_body,
        grid=(N_PAIRS // PBLK,),
        in_specs=[
            pl.BlockSpec((PBLK, N_DIST), lambda i: (i, 0)),
            pl.BlockSpec((PBLK, 32), lambda i: (i, 0)),
            _full((N_DIST, 64)),
            _full((1, 64)),
            _full((32, 64)),
            _full((1, 64)),
            _full((64, 32)),
        ],
        out_specs=pl.BlockSpec((PBLK, 32), lambda i: (i, 0)),
        out_shape=jax.ShapeDtypeStruct((N_PAIRS, 32), F32),
    )(distance, g, wdf, bdf, wcf, bcf, wfc)


def _tc_update(p, x, wcf, bcf, bdf, wfc):
    return pl.pallas_call(
        _tc_update_body,
        grid=(N_ATOMS // ABLK,),
        in_specs=[
            pl.BlockSpec((ABLK, 32), lambda i: (i, 0)),
            pl.BlockSpec((ABLK, 32), lambda i: (i, 0)),
            _full((32, 64)),
            _full((1, 64)),
            _full((1, 64)),
            _full((64, 32)),
        ],
        out_specs=pl.BlockSpec((ABLK, 32), lambda i: (i, 0)),
        out_shape=jax.ShapeDtypeStruct((N_ATOMS, 32), F32),
    )(p, x, wcf, bcf, bdf, wfc)


def _tc_head(p, x, wcf, bcf, bdf, wfc, wg0, bg0, wg1, bg1):
    return pl.pallas_call(
        _tc_head_body,
        grid=(N_ATOMS // ABLK,),
        in_specs=[
            pl.BlockSpec((ABLK, 32), lambda i: (i, 0)),
            pl.BlockSpec((ABLK, 32), lambda i: (i, 0)),
            _full((32, 64)),
            _full((1, 64)),
            _full((1, 64)),
            _full((64, 32)),
            _full((32, 128)),
            _full((1, 128)),
            _full((128, 16)),
            _full((1, 16)),
        ],
        out_specs=pl.BlockSpec((ABLK, 16), lambda i: (i, 0)),
        out_shape=jax.ShapeDtypeStruct((N_ATOMS, 16), F32),
    )(p, x, wcf, bcf, bdf, wfc, wg0, bg0, wg1, bg1)


def _tc_final(pm, sel, b):
    return pl.pallas_call(
        _tc_final_body,
        in_specs=[_full((NC, N_MOLS, 16)), _full((16, 1)), _full((1, 1))],
        out_specs=_full((N_MOLS, 1)),
        out_shape=jax.ShapeDtypeStruct((N_MOLS, 1), F32),
    )(pm, sel, b)


def kernel(atom_number, distance, atom_membership, distance_membership_i,
           distance_membership_j, embedding, W_cf, W_df, W_fc, b_cf, b_df,
           Wg0, bg0, Wg1, bg1, W_lin, b_lin):
    emb_p = jnp.zeros((32, 32), F32).at[:30, :30].set(embedding)
    wcf_p = jnp.zeros((32, 64), F32).at[:30, :60].set(W_cf)
    wdf_p = jnp.zeros((N_DIST, 64), F32).at[:, :60].set(W_df)
    wfc_p = jnp.zeros((64, 32), F32).at[:60, :30].set(W_fc)
    bcf_p = jnp.zeros((1, 64), F32).at[0, :60].set(b_cf)
    bdf_p = jnp.zeros((1, 64), F32).at[0, :60].set(b_df)
    wg0_p = jnp.zeros((32, 128), F32).at[:30, :100].set(Wg0)
    bg0_p = jnp.zeros((1, 128), F32).at[0, :100].set(bg0)
    wg1_p = jnp.zeros((128, 16), F32).at[:100, :1].set(Wg1)
    bg1_p = jnp.zeros((1, 16), F32).at[0, :1].set(bg1)
    sel = jnp.zeros((16, 1), F32).at[0, 0].set(W_lin[0, 0])
    b_fin = b_lin.reshape(1, 1).astype(F32)

    an3 = atom_number.astype(jnp.int32).reshape(N_ATOMS // ABLK, 1, ABLK)
    dmj_r = jnp.pad(
        distance_membership_j.astype(jnp.int32).reshape(NCHUNK, 1, CH),
        ((0, IDXPAD - NCHUNK), (0, 0), (0, 0)),
    )
    dmi_r = jnp.pad(
        distance_membership_i.astype(jnp.int32).reshape(NCHUNK, 1, CH),
        ((0, IDXPAD - NCHUNK), (0, 0), (0, 0)),
    )
    am_pad = jnp.pad(
        atom_membership.astype(jnp.int32), (0, ACHUNK * CH - N_ATOMS),
        constant_values=SENT,
    ).reshape(ACHUNK, 1, CH)
    am_r = jnp.pad(am_pad, ((0, 400 - ACHUNK), (0, 0), (0, 0)))

    x1 = _tc_x1(an3, emb_p)

    g1 = _sc_gather32(x1, dmj_r)
    u1 = _tc_pairs(distance, g1, wdf_p, bdf_p, wcf_p, bcf_p, wfc_p)
    s1 = _sc_segsum32(u1, dmi_r)
    x2 = _tc_update(s1, x1, wcf_p, bcf_p, bdf_p, wfc_p)

    g2 = _sc_gather32(x2, dmj_r)
    u2 = _tc_pairs(distance, g2, wdf_p, bdf_p, wcf_p, bcf_p, wfc_p)
    s2 = _sc_segsum32(u2, dmi_r)
    h2 = _tc_head(s2, x2, wcf_p, bcf_p, bdf_p, wfc_p,
                  wg0_p, bg0_p, wg1_p, bg1_p)

    pm = _sc_molsum16(h2, am_r)
    return _tc_final(pm, sel, b_fin)


# speedup vs baseline: 2.8259x; 1.1191x over previous
import functools

import jax
import jax.numpy as jnp
from jax import lax
from jax.experimental import pallas as pl
from jax.experimental.pallas import tpu as pltpu
from jax.experimental.pallas import tpu_sc as plsc

N_ATOMS = 50000
N_PAIRS = 800000
N_MOLS = 2500
N_DIST = 100

F32 = jnp.float32

NC = 2
NS = 16
NW = NC * NS

CH = 128
NCHUNK = N_PAIRS // CH
NBUF = 8
IDXPAD = 6256
KMAX_G = (NCHUNK + NW - 1) // NW
KMAX_S = (NCHUNK + NS - 1) // NS
HALF_A = N_ATOMS // NC
ACHUNK = 391
SENT = 0x7FFFFFFF

PBLK = 4000
ABLK = 2000

_CP_SC = pltpu.CompilerParams(use_tc_tiling_on_sc=False)


def _mesh():
    return plsc.VectorSubcoreMesh(core_axis_name="c", subcore_axis_name="s")


GMAX_G = (NCHUNK + NW * NBUF - 1) // (NW * NBUF)


@functools.partial(
    pl.kernel,
    out_type=jax.ShapeDtypeStruct((N_PAIRS, 32), F32),
    mesh=_mesh(),
    compiler_params=_CP_SC,
    scratch_types=[
        [pltpu.VMEM((1, CH), jnp.int32)] * NBUF,
        pltpu.VMEM((NBUF * CH, 32), F32),
        pltpu.SemaphoreType.DMA,
        pltpu.SemaphoreType.DMA,
        pltpu.SemaphoreType.DMA,
    ],
)
def _sc_gather32(table_hbm, idx_hbm, out_hbm, idxbuf, rowbuf, si, sg, so):
    wid = lax.axis_index("c") * NS + lax.axis_index("s")

    @pl.loop(0, GMAX_G)
    def _(g):
        chs = [wid + (g * NBUF + b) * NW for b in range(NBUF)]

        def _idx_copy(b):
            return pltpu.make_async_copy(idx_hbm.at[chs[b]], idxbuf[b], si)

        def _gat_copy(b):
            return pltpu.make_async_copy(
                table_hbm.at[idxbuf[b].at[0]],
                rowbuf.at[pl.ds(b * CH, CH)],
                sg,
            )

        def _out_copy(b):
            return pltpu.make_async_copy(
                rowbuf.at[pl.ds(b * CH, CH)],
                out_hbm.at[pl.ds(chs[b] * CH, CH)],
                so,
            )

        def _staged(mk):
            for b in range(NBUF):
                @pl.when(chs[b] < NCHUNK)
                def _(b=b):
                    mk(b).start()
            for b in range(NBUF):
                @pl.when(chs[b] < NCHUNK)
                def _(b=b):
                    mk(b).wait()

        _staged(_idx_copy)
        _staged(_gat_copy)
        _staged(_out_copy)


GMAX_S = (NCHUNK + NS * NBUF - 1) // (NS * NBUF)


@functools.partial(
    pl.kernel,
    out_type=jax.ShapeDtypeStruct((N_ATOMS, 32), F32),
    mesh=_mesh(),
    compiler_params=_CP_SC,
    scratch_types=[
        [pltpu.VMEM((1, CH), jnp.int32)] * NBUF,
        [pltpu.VMEM((1, CH), jnp.int32)] * NBUF,
        pltpu.VMEM((NBUF * CH, 32), F32),
        pltpu.VMEM((1000, 32), F32),
        pltpu.VMEM_SHARED((HALF_A, 32), F32),
        pltpu.SemaphoreType.DMA,
        pltpu.SemaphoreType.DMA,
        pltpu.SemaphoreType.DMA,
    ],
)
def _sc_segsum32(u_hbm, idx_hbm, out_hbm, idxbuf, idxbuf2, ubuf, bigbuf,
                 table, si, su, sa):
    c_ax = lax.axis_index("c")
    sid = lax.axis_index("s")
    base = c_ax * HALF_A

    @pl.loop(0, 1000)
    def _(r):
        bigbuf[r, pl.ds(0, 16)] = jnp.zeros((16,), F32)
        bigbuf[r, pl.ds(16, 16)] = jnp.zeros((16,), F32)

    @pl.loop(0, 2)
    def _(kk):
        ch = sid + kk * NS

        @pl.when(ch < HALF_A // 1000)
        def _():
            pltpu.sync_copy(bigbuf, table.at[pl.ds(ch * 1000, 1000)])

    plsc.subcore_barrier()

    @pl.loop(0, GMAX_S)
    def _(g):
        chs = [sid + (g * NBUF + b) * NS for b in range(NBUF)]

        def _idx_copy(b):
            return pltpu.make_async_copy(idx_hbm.at[chs[b]], idxbuf[b], si)

        def _u_copy(b):
            return pltpu.make_async_copy(
                u_hbm.at[pl.ds(chs[b] * CH, CH)],
                ubuf.at[pl.ds(b * CH, CH)],
                su,
            )

        scat_descs = [None] * NBUF

        def _scat_start(b):
            scat_descs[b] = pltpu.async_copy(
                ubuf.at[pl.ds(b * CH, CH)],
                table.at[plsc.Indices(idxbuf2[b].at[0], ignored_value=SENT)],
                sa,
                add=True,
            )

        for b in range(NBUF):
            @pl.when(chs[b] < NCHUNK)
            def _(b=b):
                _idx_copy(b).start()
                _u_copy(b).start()
        for b in range(NBUF):
            @pl.when(chs[b] < NCHUNK)
            def _(b=b):
                _idx_copy(b).wait()
        for b in range(NBUF):
            @pl.when(chs[b] < NCHUNK)
            def _(b=b):
                for j in range(CH // 16):
                    v = idxbuf[b][0, pl.ds(j * 16, 16)] - base
                    ok = (v >= 0) & (v < HALF_A)
                    idxbuf2[b][0, pl.ds(j * 16, 16)] = jnp.where(ok, v, SENT)
        for b in range(NBUF):
            @pl.when(chs[b] < NCHUNK)
            def _(b=b):
                _u_copy(b).wait()
        for b in range(NBUF):
            @pl.when(chs[b] < NCHUNK)
            def _(b=b):
                _scat_start(b)
        for b in range(NBUF):
            @pl.when(chs[b] < NCHUNK)
            def _(b=b):
                scat_descs[b].wait()

    plsc.subcore_barrier()

    @pl.loop(0, 2)
    def _(kk):
        ch = sid + kk * NS

        @pl.when(ch < HALF_A // 1000)
        def _():
            pltpu.sync_copy(table.at[pl.ds(ch * 1000, 1000)], bigbuf)
            pltpu.sync_copy(bigbuf, out_hbm.at[pl.ds(base + ch * 1000, 1000)])


@functools.partial(
    pl.kernel,
    out_type=jax.ShapeDtypeStruct((NC, N_MOLS, 16), F32),
    mesh=_mesh(),
    compiler_params=_CP_SC,
    scratch_types=[
        pltpu.VMEM((1, CH), jnp.int32),
        pltpu.VMEM((CH, 16), F32),
        pltpu.VMEM((1000, 16), F32),
        pltpu.VMEM_SHARED((N_MOLS, 16), F32),
    ],
)
def _sc_molsum16(h_hbm, idx_hbm, out_hbm, idxbuf, ubuf, bigbuf, table):
    c_ax = lax.axis_index("c")
    sid = lax.axis_index("s")

    @pl.loop(0, 1000)
    def _(r):
        bigbuf[r, pl.ds(0, 16)] = jnp.zeros((16,), F32)

    @pl.loop(0, 3)
    def _(kk):
        ch = sid + kk * NS

        @pl.when(ch < 2)
        def _():
            pltpu.sync_copy(bigbuf, table.at[pl.ds(ch * 1000, 1000)])

        @pl.when(ch == 2)
        def _():
            pltpu.sync_copy(
                bigbuf.at[pl.ds(0, 500)], table.at[pl.ds(2000, 500)]
            )

    plsc.subcore_barrier()

    @pl.loop(0, (ACHUNK + NW - 1) // NW + 1)
    def _(kk):
        ch = (sid + kk * NS) * NC + c_ax

        @pl.when(ch < ACHUNK - 1)
        def _():
            pltpu.sync_copy(idx_hbm.at[ch], idxbuf)
            pltpu.sync_copy(h_hbm.at[pl.ds(ch * CH, CH)], ubuf)
            pltpu.sync_copy(
                ubuf,
                table.at[plsc.Indices(idxbuf.at[0], ignored_value=SENT)],
                add=True,
            )

        @pl.when(ch == ACHUNK - 1)
        def _():
            pltpu.sync_copy(idx_hbm.at[ch], idxbuf)
            pltpu.sync_copy(
                h_hbm.at[pl.ds(ch * CH, 80)], ubuf.at[pl.ds(0, 80)]
            )
            pltpu.sync_copy(
                ubuf,
                table.at[plsc.Indices(idxbuf.at[0], ignored_value=SENT)],
                add=True,
            )

    plsc.subcore_barrier()

    @pl.loop(0, 3)
    def _(kk):
        ch = sid + kk * NS

        @pl.when(ch < 2)
        def _():
            pltpu.sync_copy(table.at[pl.ds(ch * 1000, 1000)], bigbuf)
            pltpu.sync_copy(bigbuf, out_hbm.at[c_ax, pl.ds(ch * 1000, 1000)])

        @pl.when(ch == 2)
        def _():
            pltpu.sync_copy(
                table.at[pl.ds(2000, 500)], bigbuf.at[pl.ds(0, 500)]
            )
            pltpu.sync_copy(
                bigbuf.at[pl.ds(0, 500)], out_hbm.at[c_ax, pl.ds(2000, 500)]
            )


def _tc_x1_body(an_ref, emb_ref, out_ref):
    an = an_ref[0, 0, :]
    oh = (
        lax.broadcasted_iota(jnp.int32, (ABLK, 32), 1) == an[:, None]
    ).astype(F32)
    out_ref[...] = jnp.dot(oh, emb_ref[...], preferred_element_type=F32)


def _tc_pairs_body(d_ref, g_ref, wdf_ref, bdf_ref, wcf_ref, bcf_ref, wfc_ref, out_ref):
    dh = jnp.dot(d_ref[...], wdf_ref[...], preferred_element_type=F32) + bdf_ref[...]
    afh = jnp.dot(g_ref[...], wcf_ref[...], preferred_element_type=F32) + bcf_ref[...]
    out_ref[...] = jnp.tanh(
        jnp.dot(dh * afh, wfc_ref[...], preferred_element_type=F32)
    )


def _tc_update_body(p_ref, x_ref, wcf_ref, bcf_ref, bdf_ref, wfc_ref, out_ref):
    x = x_ref[...]
    afh = jnp.dot(x, wcf_ref[...], preferred_element_type=F32) + bcf_ref[...]
    oii = jnp.tanh(
        jnp.dot(bdf_ref[...] * afh, wfc_ref[...], preferred_element_type=F32)
    )
    out_ref[...] = p_ref[...] - oii + x


def _tc_head_body(
    p_ref, x_ref, wcf_ref, bcf_ref, bdf_ref, wfc_ref,
    wg0_ref, bg0_ref, wg1_ref, bg1_ref, out_ref,
):
    x = x_ref[...]
    afh = jnp.dot(x, wcf_ref[...], preferred_element_type=F32) + bcf_ref[...]
    oii = jnp.tanh(
        jnp.dot(bdf_ref[...] * afh, wfc_ref[...], preferred_element_type=F32)
    )
    x3 = p_ref[...] - oii + x
    h = jnp.tanh(jnp.dot(x3, wg0_ref[...], preferred_element_type=F32) + bg0_ref[...])
    out_ref[...] = jnp.tanh(
        jnp.dot(h, wg1_ref[...], preferred_element_type=F32) + bg1_ref[...]
    )


def _tc_final_body(pm_ref, sel_ref, b_ref, out_ref):
    s = pm_ref[0] + pm_ref[1]
    out_ref[...] = jnp.dot(s, sel_ref[...], preferred_element_type=F32) + b_ref[...]


def _full(shape):
    return pl.BlockSpec(shape, lambda *_: tuple(0 for _ in shape))


def _tc_x1(an3, emb_p):
    return pl.pallas_call(
        _tc_x1_body,
        grid=(N_ATOMS // ABLK,),
        in_specs=[
            pl.BlockSpec((1, 1, ABLK), lambda i: (i, 0, 0)),
            _full((32, 32)),
        ],
        out_specs=pl.BlockSpec((ABLK, 32), lambda i: (i, 0)),
        out_shape=jax.ShapeDtypeStruct((N_ATOMS, 32), F32),
    )(an3, emb_p)


def _tc_pairs(distance, g, wdf, bdf, wcf, bcf, wfc):
    return pl.pallas_call(
        _tc_pairs_body,
        grid=(N_PAIRS // PBLK,),
        in_specs=[
            pl.BlockSpec((PBLK, N_DIST), lambda i: (i, 0)),
            pl.BlockSpec((PBLK, 32), lambda i: (i, 0)),
            _full((N_DIST, 64)),
            _full((1, 64)),
            _full((32, 64)),
            _full((1, 64)),
            _full((64, 32)),
        ],
        out_specs=pl.BlockSpec((PBLK, 32), lambda i: (i, 0)),
        out_shape=jax.ShapeDtypeStruct((N_PAIRS, 32), F32),
    )(distance, g, wdf, bdf, wcf, bcf, wfc)


def _tc_update(p, x, wcf, bcf, bdf, wfc):
    return pl.pallas_call(
        _tc_update_body,
        grid=(N_ATOMS // ABLK,),
        in_specs=[
            pl.BlockSpec((ABLK, 32), lambda i: (i, 0)),
            pl.BlockSpec((ABLK, 32), lambda i: (i, 0)),
            _full((32, 64)),
            _full((1, 64)),
            _full((1, 64)),
            _full((64, 32)),
        ],
        out_specs=pl.BlockSpec((ABLK, 32), lambda i: (i, 0)),
        out_shape=jax.ShapeDtypeStruct((N_ATOMS, 32), F32),
    )(p, x, wcf, bcf, bdf, wfc)


def _tc_head(p, x, wcf, bcf, bdf, wfc, wg0, bg0, wg1, bg1):
    return pl.pallas_call(
        _tc_head_body,
        grid=(N_ATOMS // ABLK,),
        in_specs=[
            pl.BlockSpec((ABLK, 32), lambda i: (i, 0)),
            pl.BlockSpec((ABLK, 32), lambda i: (i, 0)),
            _full((32, 64)),
            _full((1, 64)),
            _full((1, 64)),
            _full((64, 32)),
            _full((32, 128)),
            _full((1, 128)),
            _full((128, 16)),
            _full((1, 16)),
        ],
        out_specs=pl.BlockSpec((ABLK, 16), lambda i: (i, 0)),
        out_shape=jax.ShapeDtypeStruct((N_ATOMS, 16), F32),
    )(p, x, wcf, bcf, bdf, wfc, wg0, bg0, wg1, bg1)


def _tc_final(pm, sel, b):
    return pl.pallas_call(
        _tc_final_body,
        in_specs=[_full((NC, N_MOLS, 16)), _full((16, 1)), _full((1, 1))],
        out_specs=_full((N_MOLS, 1)),
        out_shape=jax.ShapeDtypeStruct((N_MOLS, 1), F32),
    )(pm, sel, b)


def kernel(atom_number, distance, atom_membership, distance_membership_i,
           distance_membership_j, embedding, W_cf, W_df, W_fc, b_cf, b_df,
           Wg0, bg0, Wg1, bg1, W_lin, b_lin):
    emb_p = jnp.zeros((32, 32), F32).at[:30, :30].set(embedding)
    wcf_p = jnp.zeros((32, 64), F32).at[:30, :60].set(W_cf)
    wdf_p = jnp.zeros((N_DIST, 64), F32).at[:, :60].set(W_df)
    wfc_p = jnp.zeros((64, 32), F32).at[:60, :30].set(W_fc)
    bcf_p = jnp.zeros((1, 64), F32).at[0, :60].set(b_cf)
    bdf_p = jnp.zeros((1, 64), F32).at[0, :60].set(b_df)
    wg0_p = jnp.zeros((32, 128), F32).at[:30, :100].set(Wg0)
    bg0_p = jnp.zeros((1, 128), F32).at[0, :100].set(bg0)
    wg1_p = jnp.zeros((128, 16), F32).at[:100, :1].set(Wg1)
    bg1_p = jnp.zeros((1, 16), F32).at[0, :1].set(bg1)
    sel = jnp.zeros((16, 1), F32).at[0, 0].set(W_lin[0, 0])
    b_fin = b_lin.reshape(1, 1).astype(F32)

    an3 = atom_number.astype(jnp.int32).reshape(N_ATOMS // ABLK, 1, ABLK)
    dmj_r = jnp.pad(
        distance_membership_j.astype(jnp.int32).reshape(NCHUNK, 1, CH),
        ((0, IDXPAD - NCHUNK), (0, 0), (0, 0)),
    )
    dmi_r = jnp.pad(
        distance_membership_i.astype(jnp.int32).reshape(NCHUNK, 1, CH),
        ((0, IDXPAD - NCHUNK), (0, 0), (0, 0)),
    )
    am_pad = jnp.pad(
        atom_membership.astype(jnp.int32), (0, ACHUNK * CH - N_ATOMS),
        constant_values=SENT,
    ).reshape(ACHUNK, 1, CH)
    am_r = jnp.pad(am_pad, ((0, 400 - ACHUNK), (0, 0), (0, 0)))

    x1 = _tc_x1(an3, emb_p)

    g1 = _sc_gather32(x1, dmj_r)
    u1 = _tc_pairs(distance, g1, wdf_p, bdf_p, wcf_p, bcf_p, wfc_p)
    s1 = _sc_segsum32(u1, dmi_r)
    x2 = _tc_update(s1, x1, wcf_p, bcf_p, bdf_p, wfc_p)

    g2 = _sc_gather32(x2, dmj_r)
    u2 = _tc_pairs(distance, g2, wdf_p, bdf_p, wcf_p, bcf_p, wfc_p)
    s2 = _sc_segsum32(u2, dmi_r)
    h2 = _tc_head(s2, x2, wcf_p, bcf_p, bdf_p, wfc_p,
                  wg0_p, bg0_p, wg1_p, bg1_p)

    pm = _sc_molsum16(h2, am_r)
    return _tc_final(pm, sel, b_fin)
